# baseline (speedup 1.0000x reference)
import functools

import jax
import jax.numpy as jnp
from jax import lax
from jax.experimental import pallas as pl
from jax.experimental.pallas import tpu as pltpu
from jax.experimental.pallas import tpu_sc as plsc

VOCAB = 1_000_000
D = 20
PAD_IDX = 4

_info = plsc.get_sparse_core_info()
NC, NS, L = _info.num_cores, _info.num_subcores, _info.num_lanes
NW = NC * NS

B_TOTAL = 16384 * 30
B_PER_W = B_TOTAL // NW
C = 1024
N_CHUNKS = B_PER_W // C
IDX_ROWS = C // 128
G_STREAMS = 2 * C // 128
X_ROWS_PER_W = B_PER_W // 128


def _body(x_hbm, tbl16_hbm, out_hbm, idx_v, dma_idx_v, pad_v, outb_v, sem):
    wid = lax.axis_index("s") * NC + lax.axis_index("c")
    lane = lax.iota(jnp.int32, L)
    lane2 = lane * 2
    zeros = jnp.zeros((L,), jnp.float32)
    ones = jnp.ones((L,), jnp.float32)

    def chunk(g, _):
        xrow0 = wid * X_ROWS_PER_W + g * IDX_ROWS
        base = wid * B_PER_W + g * C

        pltpu.sync_copy(x_hbm.at[pl.ds(xrow0, IDX_ROWS)], idx_v)

        def build(k, _):
            xr = idx_v[k // 8, pl.ds((k % 8) * L, L)]
            j0 = (xr * 5) >> 2
            row = jnp.full((L,), 0, jnp.int32) + (k >> 2)
            col0 = (k % 4) * 32 + lane2
            plsc.store_scatter(dma_idx_v, [row, col0], j0)
            plsc.store_scatter(dma_idx_v, [row, col0 + 1], j0 + 1)
            return _
        lax.fori_loop(0, C // L, build, None, unroll=False)

        copies = [
            pltpu.async_copy(
                tbl16_hbm.at[dma_idx_v.at[j]],
                pad_v.at[pl.ds(j * 128, 128)],
                sem,
            )
            for j in range(G_STREAMS)
        ]
        for cp in copies:
            cp.wait()

        def compact(k, _):
            xr = idx_v[k // 8, pl.ds((k % 8) * L, L)]
            off = (xr & 3) << 2
            mf = jnp.where(xr == PAD_IDX, zeros, ones)
            r = k * L + lane
            r2 = r * 2
            for c in range(D):
                t = off + c
                b = t >> 4
                row = r2 + b
                col = t - (b << 4)
                val = plsc.load_gather(pad_v, [row, col]) * mf
                plsc.store_scatter(outb_v, [r, jnp.full((L,), c, jnp.int32)], val)
            return _
        lax.fori_loop(0, C // L, compact, None, unroll=False)

        pltpu.sync_copy(outb_v, out_hbm.at[pl.ds(base, C)])
        return _

    lax.fori_loop(0, N_CHUNKS, chunk, None, unroll=False)


@functools.partial(jax.jit, static_argnames=())
def kernel(x, table):
    x2 = x.reshape(-1, 128)
    tbl16 = table.reshape(VOCAB * D // 16, 16)
    mesh = plsc.VectorSubcoreMesh(core_axis_name="c", subcore_axis_name="s")
    out = pl.kernel(
        _body,
        out_type=jax.ShapeDtypeStruct((B_TOTAL, D), jnp.float32),
        mesh=mesh,
        scratch_types=[
            pltpu.VMEM((IDX_ROWS, 128), jnp.int32),
            pltpu.VMEM((G_STREAMS, 128), jnp.int32),
            pltpu.VMEM((2 * C, 16), jnp.float32),
            pltpu.VMEM((C, D), jnp.float32),
            pltpu.SemaphoreType.DMA,
        ],
        compiler_params=pltpu.CompilerParams(
            use_tc_tiling_on_sc=False, needs_layout_passes=False
        ),
    )(x2, tbl16)
    return out.reshape(16384, 30, D)
